# Initial kernel scaffold; baseline (speedup 1.0000x reference)
import jax
import jax.numpy as jnp
from jax.experimental import pallas as pl


def kernel(x, edge_index, W1, b1, W2, b2, W3, b3):
    raise NotImplementedError("write your pallas kernel here")


# speedup vs baseline: 9.2165x; 9.2165x over previous
import functools

import jax
import jax.numpy as jnp
from jax import lax
from jax.experimental import pallas as pl
from jax.experimental.pallas import tpu as pltpu
from jax.experimental.pallas import tpu_sc as plsc

N = 10000
D = 128
N_PAD = 10240
PAD_ROW = N_PAD - 1
K = 128

_info = plsc.get_sparse_core_info()
NC = _info.num_cores
NS = _info.num_subcores
NW = NC * NS
RPS = N_PAD // NS

_mesh = plsc.VectorSubcoreMesh(core_axis_name="c", subcore_axis_name="s")


def _make_deg(epw, n_chunks):
    @functools.partial(
        pl.kernel,
        mesh=_mesh,
        out_type=jax.ShapeDtypeStruct((NC * N_PAD,), jnp.float32),
        scratch_types=[
            pltpu.VMEM((K,), jnp.int32),
            pltpu.VMEM((K,), jnp.float32),
            pltpu.VMEM((K,), jnp.float32),
            pltpu.VMEM_SHARED((N_PAD,), jnp.float32),
        ],
    )
    def deg_kernel(dst_hbm, out_hbm, idx_v, ones_v, zero_v, acc_sh):
        c = lax.axis_index("c")
        s = lax.axis_index("s")
        wid = c * NS + s

        def initbuf(i, carry):
            ones_v[pl.ds(i * 16, 16)] = jnp.ones((16,), jnp.float32)
            zero_v[pl.ds(i * 16, 16)] = jnp.zeros((16,), jnp.float32)
            return carry

        lax.fori_loop(0, K // 16, initbuf, 0)

        for j in range(RPS // K):
            off = pl.multiple_of(s * RPS + j * K, 8)
            pltpu.sync_copy(zero_v, acc_sh.at[pl.ds(off, K)])
        plsc.subcore_barrier()

        base = wid * epw

        def chunk(i, carry):
            off = pl.multiple_of(base + i * K, 8)
            pltpu.sync_copy(dst_hbm.at[pl.ds(off, K)], idx_v)
            pltpu.sync_copy(ones_v, acc_sh.at[idx_v], add=True)
            return carry

        lax.fori_loop(0, n_chunks, chunk, 0)
        plsc.subcore_barrier()

        src_off = pl.multiple_of(s * RPS, 8)
        dst_off = pl.multiple_of(c * N_PAD + s * RPS, 8)
        pltpu.sync_copy(acc_sh.at[pl.ds(src_off, RPS)],
                        out_hbm.at[pl.ds(dst_off, RPS)])

    return deg_kernel


def _make_agg(epw, n_chunks):
    @functools.partial(
        pl.kernel,
        mesh=_mesh,
        out_type=jax.ShapeDtypeStruct((NC * N_PAD, D), jnp.float32),
        scratch_types=[
            pltpu.VMEM((K,), jnp.int32),
            pltpu.VMEM((K,), jnp.int32),
            pltpu.VMEM((K, D), jnp.float32),
            pltpu.VMEM_SHARED((N_PAD, D), jnp.float32),
            pltpu.SemaphoreType.DMA,
        ],
    )
    def agg_kernel(g_hbm, src_hbm, dst_hbm, out_hbm, src_v, dst_v, rows_v, acc_sh, sem):
        c = lax.axis_index("c")
        s = lax.axis_index("s")
        wid = c * NS + s

        def zrow(i, carry):
            rows_v[i // (D // 16), pl.ds((i % (D // 16)) * 16, 16)] = (
                jnp.zeros((16,), jnp.float32))
            return carry

        lax.fori_loop(0, K * D // 16, zrow, 0)
        for j in range(RPS // K):
            pltpu.sync_copy(rows_v, acc_sh.at[pl.ds(s * RPS + j * K, K)])
        plsc.subcore_barrier()

        base = wid * epw

        def chunk(i, carry):
            off = pl.multiple_of(base + i * K, 8)
            pltpu.sync_copy(src_hbm.at[pl.ds(off, K)], src_v)
            pltpu.sync_copy(dst_hbm.at[pl.ds(off, K)], dst_v)
            pltpu.async_copy(g_hbm.at[src_v], rows_v, sem).wait()
            pltpu.sync_copy(rows_v, acc_sh.at[dst_v], add=True)
            return carry

        lax.fori_loop(0, n_chunks, chunk, 0)
        plsc.subcore_barrier()

        pltpu.sync_copy(acc_sh.at[pl.ds(s * RPS, RPS)],
                        out_hbm.at[pl.ds(c * N_PAD + s * RPS, RPS)])

    return agg_kernel


_BR = 1024
_NBLK = N_PAD // _BR


def _tc_first(x_pad, W1, dT):
    def body(x_ref, w_ref, dT_ref, g_ref, dinv_ref):
        dsum = dT_ref[:, 0:1] + dT_ref[:, 1:2]
        dinv = lax.rsqrt(1.0 + dsum)
        g_ref[...] = dinv * jnp.dot(x_ref[...], w_ref[...],
                                    preferred_element_type=jnp.float32)
        dinv_ref[...] = dinv

    return pl.pallas_call(
        body,
        grid=(_NBLK,),
        in_specs=[
            pl.BlockSpec((_BR, D), lambda i: (i, 0)),
            pl.BlockSpec((D, D), lambda i: (0, 0)),
            pl.BlockSpec((_BR, 2), lambda i: (i, 0)),
        ],
        out_specs=[
            pl.BlockSpec((_BR, D), lambda i: (i, 0)),
            pl.BlockSpec((_BR, 1), lambda i: (i, 0)),
        ],
        out_shape=[
            jax.ShapeDtypeStruct((N_PAD, D), jnp.float32),
            jax.ShapeDtypeStruct((N_PAD, 1), jnp.float32),
        ],
    )(x_pad, W1, dT)


def _tc_mid(p, g, dinv, b, W):
    def body(p0_ref, p1_ref, g_ref, dinv_ref, b_ref, w_ref, out_ref):
        t = dinv_ref[...] * (p0_ref[...] + p1_ref[...] + g_ref[...]) + b_ref[...]
        h = jnp.maximum(t, 0.0)
        out_ref[...] = dinv_ref[...] * jnp.dot(h, w_ref[...],
                                               preferred_element_type=jnp.float32)

    return pl.pallas_call(
        body,
        grid=(_NBLK,),
        in_specs=[
            pl.BlockSpec((_BR, D), lambda i: (i, 0)),
            pl.BlockSpec((_BR, D), lambda i: (i + _NBLK, 0)),
            pl.BlockSpec((_BR, D), lambda i: (i, 0)),
            pl.BlockSpec((_BR, 1), lambda i: (i, 0)),
            pl.BlockSpec((1, D), lambda i: (0, 0)),
            pl.BlockSpec((D, D), lambda i: (0, 0)),
        ],
        out_specs=pl.BlockSpec((_BR, D), lambda i: (i, 0)),
        out_shape=jax.ShapeDtypeStruct((N_PAD, D), jnp.float32),
    )(p, p, g, dinv, b, W)


def _tc_last(p, g, dinv, b):
    def body(p0_ref, p1_ref, g_ref, dinv_ref, b_ref, out_ref):
        t = dinv_ref[...] * (p0_ref[...] + p1_ref[...] + g_ref[...]) + b_ref[...]
        out_ref[...] = jnp.maximum(t, 0.0)

    return pl.pallas_call(
        body,
        grid=(-(-N // _BR),),
        in_specs=[
            pl.BlockSpec((_BR, D), lambda i: (i, 0)),
            pl.BlockSpec((_BR, D), lambda i: (i + _NBLK, 0)),
            pl.BlockSpec((_BR, D), lambda i: (i, 0)),
            pl.BlockSpec((_BR, 1), lambda i: (i, 0)),
            pl.BlockSpec((1, D), lambda i: (0, 0)),
        ],
        out_specs=pl.BlockSpec((_BR, D), lambda i: (i, 0)),
        out_shape=jax.ShapeDtypeStruct((N, D), jnp.float32),
    )(p, p, g, dinv, b)


def kernel(x, edge_index, W1, b1, W2, b2, W3, b3):
    E = edge_index.shape[1]
    n_chunks = -(-E // (NW * K))
    epw = n_chunks * K
    e_pad = epw * NW

    src = edge_index[0]
    dst = edge_index[1]
    if e_pad > E:
        fill = jnp.full((e_pad - E,), PAD_ROW, dtype=jnp.int32)
        src = jnp.concatenate([src, fill])
        dst = jnp.concatenate([dst, fill])

    x_pad = jnp.zeros((N_PAD, D), jnp.float32).at[:N, :].set(x)

    deg_fn = _make_deg(epw, n_chunks)
    agg_fn = _make_agg(epw, n_chunks)

    d = deg_fn(dst)
    dT = jnp.stack([d[:N_PAD], d[N_PAD:]], axis=1)

    g1, dinv = _tc_first(x_pad, W1, dT)
    p1 = agg_fn(g1, src, dst)
    g2 = _tc_mid(p1, g1, dinv, b1.reshape(1, D), W2)
    p2 = agg_fn(g2, src, dst)
    g3 = _tc_mid(p2, g2, dinv, b2.reshape(1, D), W3)
    p3 = agg_fn(g3, src, dst)
    return _tc_last(p3, g3, dinv, b3.reshape(1, D))
